# baseline (speedup 1.0000x reference)
import jax
import jax.numpy as jnp
from jax.experimental import pallas as pl
from jax.experimental.pallas import tpu as pltpu


def _body(x2_ref, w1p_ref, w2p_ref, inc_ref, out_ref,
          m1e_ref, blkb_ref, m2e_ref, acc_ref):
    i = pl.program_id(0)
    nblocks = pl.num_programs(0) - 1
    slot = jax.lax.rem(i, 2)
    prev = jax.lax.rem(i + 1, 2)

    @pl.when(i == 0)
    def _():
        m1p = jnp.dot(x2_ref[...], w1p_ref[...], preferred_element_type=jnp.float32)
        col = jax.lax.broadcasted_iota(jnp.int32, m1p.shape, 1)
        m1e_ref[...] = jnp.where(col == 64, 1.0, m1p).astype(jnp.bfloat16)

    @pl.when(i < nblocks)
    def _():
        blk = inc_ref[...].astype(jnp.bfloat16)
        blkb_ref[slot] = blk
        y1e = jnp.dot(blk, m1e_ref[...], preferred_element_type=jnp.float32)
        y1 = y1e[:, :64]
        rs = y1e[:, 64:65]
        x1_blk = jax.nn.sigmoid(y1 * (1.0 / rs))
        m2p = jnp.dot(x1_blk, w2p_ref[...], preferred_element_type=jnp.float32)
        col = jax.lax.broadcasted_iota(jnp.int32, m2p.shape, 1)
        m2e_ref[slot] = jnp.where(col == 64, 1.0, m2p).astype(jnp.bfloat16)

    @pl.when(i > 0)
    def _():
        contrib = jax.lax.dot_general(
            m2e_ref[prev], blkb_ref[prev], (((0,), (0,)), ((), ())),
            preferred_element_type=jnp.float32,
        )

        @pl.when(i == 1)
        def _():
            acc_ref[...] = contrib

        @pl.when(i > 1)
        def _():
            acc_ref[...] += contrib

    @pl.when(i == nblocks)
    def _():
        y2 = acc_ref[:64, :]
        cs = acc_ref[64:65, :]
        out_ref[...] = jnp.transpose(jax.nn.sigmoid(y2 * (1.0 / cs)))


def kernel(x_2, incidence_2, w1, w2):
    n_edges, n_faces = incidence_2.shape
    in_c = x_2.shape[1]
    mid_c = w1.shape[1]
    out_c = w2.shape[1]
    bm = 1024
    nblocks = n_edges // bm

    w1p = jnp.pad(w1, ((0, 0), (0, 128 - mid_c)))
    w2p = jnp.pad(w2, ((0, 0), (0, 72 - out_c)))

    out = pl.pallas_call(
        _body,
        grid=(nblocks + 1,),
        in_specs=[
            pl.BlockSpec((n_faces, in_c), lambda i: (0, 0)),
            pl.BlockSpec((in_c, 128), lambda i: (0, 0)),
            pl.BlockSpec((mid_c, 72), lambda i: (0, 0)),
            pl.BlockSpec((bm, n_faces), lambda i: (jnp.minimum(i, nblocks - 1), 0)),
        ],
        out_specs=pl.BlockSpec((n_faces, out_c), lambda i: (0, 0)),
        out_shape=jax.ShapeDtypeStruct((n_faces, out_c), jnp.float32),
        scratch_shapes=[
            pltpu.VMEM((n_faces, 128), jnp.bfloat16),
            pltpu.VMEM((2, bm, n_faces), jnp.bfloat16),
            pltpu.VMEM((2, bm, 72), jnp.bfloat16),
            pltpu.VMEM((72, n_faces), jnp.float32),
        ],
    )(x_2, w1p, w2p, incidence_2)

    return out


# speedup vs baseline: 1.0247x; 1.0247x over previous
import jax
import jax.numpy as jnp
from jax.experimental import pallas as pl
from jax.experimental.pallas import tpu as pltpu


def _body(x2_ref, w1p_ref, w2p_ref, inc_ref, out_ref,
          m1e_ref, blkb_ref, m2e_ref, acc_ref):
    i = pl.program_id(0)
    nblocks = pl.num_programs(0) - 1
    slot = jax.lax.rem(i, 2)
    prev = jax.lax.rem(i + 1, 2)

    @pl.when(i == 0)
    def _():
        m1p = jnp.dot(x2_ref[...], w1p_ref[...], preferred_element_type=jnp.float32)
        col = jax.lax.broadcasted_iota(jnp.int32, m1p.shape, 1)
        m1e_ref[...] = jnp.where(col == 64, 1.0, m1p).astype(jnp.bfloat16)

    @pl.when(i < nblocks)
    def _():
        blk = inc_ref[...].astype(jnp.bfloat16)
        blkb_ref[slot] = blk
        y1e = jnp.dot(blk, m1e_ref[...], preferred_element_type=jnp.float32)
        y1 = y1e[:, :64]
        rs = y1e[:, 64:65]
        x1_blk = jax.nn.sigmoid(y1 * (1.0 / rs))
        m2p = jnp.dot(x1_blk, w2p_ref[...], preferred_element_type=jnp.float32)
        col = jax.lax.broadcasted_iota(jnp.int32, m2p.shape, 1)
        m2e_ref[slot] = jnp.where(col == 64, 1.0, m2p).astype(jnp.bfloat16)

    @pl.when(i > 0)
    def _():
        contrib = jax.lax.dot_general(
            m2e_ref[prev], blkb_ref[prev], (((0,), (0,)), ((), ())),
            preferred_element_type=jnp.float32,
        )

        @pl.when(i == 1)
        def _():
            acc_ref[...] = contrib

        @pl.when(i > 1)
        def _():
            acc_ref[...] += contrib

    @pl.when(i == nblocks)
    def _():
        y2 = acc_ref[:64, :]
        cs = acc_ref[64:65, :]
        out_ref[...] = jnp.transpose(jax.nn.sigmoid(y2 * (1.0 / cs)))


def kernel(x_2, incidence_2, w1, w2):
    n_edges, n_faces = incidence_2.shape
    in_c = x_2.shape[1]
    mid_c = w1.shape[1]
    out_c = w2.shape[1]
    bm = 512
    nblocks = n_edges // bm

    w1p = jnp.pad(w1, ((0, 0), (0, 128 - mid_c)))
    w2p = jnp.pad(w2, ((0, 0), (0, 72 - out_c)))

    out = pl.pallas_call(
        _body,
        grid=(nblocks + 1,),
        in_specs=[
            pl.BlockSpec((n_faces, in_c), lambda i: (0, 0)),
            pl.BlockSpec((in_c, 128), lambda i: (0, 0)),
            pl.BlockSpec((mid_c, 72), lambda i: (0, 0)),
            pl.BlockSpec((bm, n_faces), lambda i: (jnp.minimum(i, nblocks - 1), 0)),
        ],
        out_specs=pl.BlockSpec((n_faces, out_c), lambda i: (0, 0)),
        out_shape=jax.ShapeDtypeStruct((n_faces, out_c), jnp.float32),
        scratch_shapes=[
            pltpu.VMEM((n_faces, 128), jnp.bfloat16),
            pltpu.VMEM((2, bm, n_faces), jnp.bfloat16),
            pltpu.VMEM((2, bm, 72), jnp.bfloat16),
            pltpu.VMEM((72, n_faces), jnp.float32),
        ],
    )(x_2, w1p, w2p, incidence_2)

    return out
